# baseline (speedup 1.0000x reference)
import jax
import jax.numpy as jnp
from jax.experimental import pallas as pl

_TN = 256


def _chamfer_body(a1_ref, a2t_ref, d1_ref, d2_ref):
    nb = pl.program_id(1)
    a1 = a1_ref[0]
    x1 = a1[:, 0:1]
    y1 = a1[:, 1:2]
    z1 = a1[:, 2:3]
    x2 = a2t_ref[0, 0:1, :]
    y2 = a2t_ref[0, 1:2, :]
    z2 = a2t_ref[0, 2:3, :]
    dx = x1 - x2
    dy = y1 - y2
    dz = z1 - z2
    d = dx * dx + dy * dy + dz * dz
    d1_ref[0] = jnp.min(d, axis=1, keepdims=True)
    colmin = jnp.min(d, axis=0, keepdims=True)

    @pl.when(nb == 0)
    def _():
        d2_ref[0] = colmin

    @pl.when(nb != 0)
    def _():
        d2_ref[0] = jnp.minimum(d2_ref[0], colmin)


def kernel(array1, array2):
    B, N, _ = array1.shape
    M = array2.shape[1]
    a2t = jnp.swapaxes(array2, 1, 2)
    nblocks = N // _TN
    d1, d2 = pl.pallas_call(
        _chamfer_body,
        grid=(B, nblocks),
        in_specs=[
            pl.BlockSpec((1, _TN, 3), lambda b, i: (b, i, 0)),
            pl.BlockSpec((1, 3, M), lambda b, i: (b, 0, 0)),
        ],
        out_specs=[
            pl.BlockSpec((1, _TN, 1), lambda b, i: (b, i, 0)),
            pl.BlockSpec((1, 1, M), lambda b, i: (b, 0, 0)),
        ],
        out_shape=[
            jax.ShapeDtypeStruct((B, N, 1), jnp.float32),
            jax.ShapeDtypeStruct((B, 1, M), jnp.float32),
        ],
    )(array1, a2t)
    return jnp.mean(d1) + jnp.mean(d2)


# speedup vs baseline: 1.5889x; 1.5889x over previous
import jax
import jax.numpy as jnp
from jax.experimental import pallas as pl

_TN = 512
_K = 8


def _chamfer_body(a_ref, b_ref, d1_ref, d2_ref):
    nb = pl.program_id(1)
    d = jnp.dot(a_ref[0], b_ref[0], preferred_element_type=jnp.float32)
    d1_ref[0] = jnp.maximum(jnp.min(d, axis=1, keepdims=True), 0.0)
    colmin = jnp.min(d, axis=0, keepdims=True)

    @pl.when(nb == 0)
    def _():
        d2_ref[0] = colmin

    @pl.when(nb != 0)
    def _():
        d2_ref[0] = jnp.minimum(d2_ref[0], colmin)


def kernel(array1, array2):
    B, N, _ = array1.shape
    M = array2.shape[1]
    f32 = jnp.float32
    n1 = jnp.sum(array1 * array1, axis=-1, keepdims=True)
    n2 = jnp.sum(array2 * array2, axis=-1, keepdims=True)
    ones1 = jnp.ones((B, N, 1), f32)
    ones2 = jnp.ones((B, M, 1), f32)
    zeros = jnp.zeros((B, N, _K - 5), f32)
    aug1 = jnp.concatenate([-2.0 * array1, n1, ones1, zeros], axis=-1)
    aug2 = jnp.concatenate([array2, ones2, n2, zeros], axis=-1)
    aug2t = jnp.swapaxes(aug2, 1, 2)
    nblocks = N // _TN
    d1, d2 = pl.pallas_call(
        _chamfer_body,
        grid=(B, nblocks),
        in_specs=[
            pl.BlockSpec((1, _TN, _K), lambda b, i: (b, i, 0)),
            pl.BlockSpec((1, _K, M), lambda b, i: (b, 0, 0)),
        ],
        out_specs=[
            pl.BlockSpec((1, _TN, 1), lambda b, i: (b, i, 0)),
            pl.BlockSpec((1, 1, M), lambda b, i: (b, 0, 0)),
        ],
        out_shape=[
            jax.ShapeDtypeStruct((B, N, 1), f32),
            jax.ShapeDtypeStruct((B, 1, M), f32),
        ],
    )(aug1, aug2t)
    d2 = jnp.maximum(d2, 0.0)
    return jnp.mean(d1) + jnp.mean(d2)


# speedup vs baseline: 1.9645x; 1.2364x over previous
import jax
import jax.numpy as jnp
from jax.experimental import pallas as pl

_TN = 2048
_K = 8


def _chamfer_body(a_ref, b_ref, d1_ref, d2_ref):
    nb = pl.program_id(1)
    d = jnp.dot(a_ref[0], b_ref[0], preferred_element_type=jnp.float32)
    d1_ref[0] = jnp.maximum(jnp.min(d, axis=1, keepdims=True), 0.0)
    colmin = jnp.min(d, axis=0, keepdims=True)

    @pl.when(nb == 0)
    def _():
        d2_ref[0] = colmin

    @pl.when(nb != 0)
    def _():
        d2_ref[0] = jnp.minimum(d2_ref[0], colmin)


def kernel(array1, array2):
    B, N, _ = array1.shape
    M = array2.shape[1]
    f32 = jnp.float32
    n1 = jnp.sum(array1 * array1, axis=-1, keepdims=True)
    n2 = jnp.sum(array2 * array2, axis=-1, keepdims=True)
    ones1 = jnp.ones((B, N, 1), f32)
    ones2 = jnp.ones((B, M, 1), f32)
    zeros = jnp.zeros((B, N, _K - 5), f32)
    aug1 = jnp.concatenate([-2.0 * array1, n1, ones1, zeros], axis=-1)
    aug2 = jnp.concatenate([array2, ones2, n2, zeros], axis=-1)
    aug2t = jnp.swapaxes(aug2, 1, 2)
    nblocks = N // _TN
    d1, d2 = pl.pallas_call(
        _chamfer_body,
        grid=(B, nblocks),
        in_specs=[
            pl.BlockSpec((1, _TN, _K), lambda b, i: (b, i, 0)),
            pl.BlockSpec((1, _K, M), lambda b, i: (b, 0, 0)),
        ],
        out_specs=[
            pl.BlockSpec((1, _TN, 1), lambda b, i: (b, i, 0)),
            pl.BlockSpec((1, 1, M), lambda b, i: (b, 0, 0)),
        ],
        out_shape=[
            jax.ShapeDtypeStruct((B, N, 1), f32),
            jax.ShapeDtypeStruct((B, 1, M), f32),
        ],
    )(aug1, aug2t)
    d2 = jnp.maximum(d2, 0.0)
    return jnp.mean(d1) + jnp.mean(d2)


# speedup vs baseline: 2.6625x; 1.3553x over previous
import jax
import jax.numpy as jnp
from jax.experimental import pallas as pl

_K = 8


def _chamfer_body(a1_ref, a2t_ref, out_ref):
    b = pl.program_id(0)
    nbatch = pl.num_programs(0)
    f32 = jnp.float32
    a1 = a1_ref[0]
    a2t = a2t_ref[0]
    n = a1.shape[0]
    m = a2t.shape[1]
    n1 = jnp.sum(a1 * a1, axis=1, keepdims=True)
    n2 = jnp.sum(a2t * a2t, axis=0, keepdims=True)
    aug1 = jnp.concatenate(
        [-2.0 * a1, n1, jnp.ones((n, 1), f32), jnp.zeros((n, _K - 5), f32)],
        axis=1)
    aug2 = jnp.concatenate(
        [a2t, jnp.ones((1, m), f32), n2, jnp.zeros((_K - 5, m), f32)],
        axis=0)
    d = jnp.dot(aug1, aug2, preferred_element_type=f32)
    rowmin = jnp.maximum(jnp.min(d, axis=1, keepdims=True), 0.0)
    colmin = jnp.maximum(jnp.min(d, axis=0, keepdims=True), 0.0)
    s = (jnp.sum(rowmin, axis=(0, 1), keepdims=True) / (nbatch * n)
         + jnp.sum(colmin, axis=(0, 1), keepdims=True) / (nbatch * m))

    @pl.when(b == 0)
    def _():
        out_ref[...] = s

    @pl.when(b != 0)
    def _():
        out_ref[...] = out_ref[...] + s


def kernel(array1, array2):
    B, N, _ = array1.shape
    M = array2.shape[1]
    a2t = jnp.swapaxes(array2, 1, 2)
    out = pl.pallas_call(
        _chamfer_body,
        grid=(B,),
        in_specs=[
            pl.BlockSpec((1, N, 3), lambda b: (b, 0, 0)),
            pl.BlockSpec((1, 3, M), lambda b: (b, 0, 0)),
        ],
        out_specs=pl.BlockSpec((1, 1), lambda b: (0, 0)),
        out_shape=jax.ShapeDtypeStruct((1, 1), jnp.float32),
    )(array1, a2t)
    return out[0, 0]
